# Initial kernel scaffold; baseline (speedup 1.0000x reference)
import jax
import jax.numpy as jnp
from jax.experimental import pallas as pl


def kernel(user_indices, item_indices, kg_adj_batch, user_table, item_table, entity_table, attn_W, attn_b, kg_W1, kg_b1, kg_W2, kg_b2, comb_W, comb_b, out_W, out_b):
    raise NotImplementedError("write your pallas kernel here")


# speedup vs baseline: 1.5953x; 1.5953x over previous
import functools

import jax
import jax.numpy as jnp
from jax import lax
from jax.experimental import pallas as pl
from jax.experimental.pallas import tpu as pltpu
from jax.experimental.pallas import tpu_sc as plsc

D = 32
NN = 50
B = 16384

_NC, _NS = 2, 16
NW = _NC * _NS

IDX_W = 128
N_IDX_ROWS = (B * NN) // IDX_W
ROWS_PER_W = N_IDX_ROWS // NW
FIRE = 8
SUPER = ROWS_PER_W // FIRE
SUPER_ROWS = FIRE * IDX_W
BPW = B // NW
UI_ROWS = BPW // IDX_W


def _sc_gather_body(adj_idx, item_idx, user_idx, entity_tab, item_tab,
                    user_tab, nb_out, item_out, user_out,
                    idx_v, rows_v, gsem):
    wid = lax.axis_index("s") * _NC + lax.axis_index("c")

    pltpu.sync_copy(adj_idx.at[pl.ds(wid * ROWS_PER_W, ROWS_PER_W)], idx_v)

    def super_chunk(sc_i, carry):
        descs = []
        for k in range(FIRE):
            d = pltpu.async_copy(
                entity_tab.at[idx_v.at[sc_i * FIRE + k]],
                rows_v.at[pl.ds(k * IDX_W, IDX_W)],
                gsem)
            descs.append(d)
        for d in descs:
            d.wait()
        pltpu.sync_copy(
            rows_v,
            nb_out.at[pl.ds(wid * ROWS_PER_W * IDX_W + sc_i * SUPER_ROWS,
                            SUPER_ROWS)])
        return carry

    lax.fori_loop(0, SUPER, super_chunk, 0)

    for idx_hbm, tab, out in ((item_idx, item_tab, item_out),
                              (user_idx, user_tab, user_out)):
        pltpu.sync_copy(idx_hbm.at[pl.ds(wid * UI_ROWS, UI_ROWS)],
                        idx_v.at[pl.ds(0, UI_ROWS)])
        descs = []
        for k in range(UI_ROWS):
            descs.append(pltpu.async_copy(
                tab.at[idx_v.at[k]],
                rows_v.at[pl.ds(k * IDX_W, IDX_W)],
                gsem))
        for d in descs:
            d.wait()
        pltpu.sync_copy(rows_v.at[pl.ds(0, BPW)],
                        out.at[pl.ds(wid * BPW, BPW)])


@functools.lru_cache(maxsize=1)
def _sc_gather_fn():
    return pl.kernel(
        _sc_gather_body,
        out_type=(
            jax.ShapeDtypeStruct((B * NN, D), jnp.float32),
            jax.ShapeDtypeStruct((B, D), jnp.float32),
            jax.ShapeDtypeStruct((B, D), jnp.float32),
        ),
        mesh=plsc.VectorSubcoreMesh(core_axis_name="c", subcore_axis_name="s",
                                    num_cores=_NC, num_subcores=_NS),
        scratch_types=(
            pltpu.VMEM((ROWS_PER_W, IDX_W), jnp.int32),
            pltpu.VMEM((SUPER_ROWS, D), jnp.float32),
            pltpu.SemaphoreType.DMA,
        ),
        compiler_params=pltpu.CompilerParams(use_tc_tiling_on_sc=False),
    )


BBLK = 512


def _tc_body(nb_ref, item_ref, user_ref, wi_ref, wn_ref, ab_ref,
             w1_ref, b1_ref, w2_ref, b2_ref, cwi_ref, cwr_ref, cb_ref,
             owu_ref, owf_ref, ob_ref, out_ref):
    nb = nb_ref[...]
    item = item_ref[...]
    user = user_ref[...]

    c = jnp.sum(item * wi_ref[...], axis=1)[None, :]
    s = jnp.sum(nb * wn_ref[...][0][None, None, :], axis=2)
    s = s + c + ab_ref[0, 0]
    s = jnp.where(s >= 0.0, s, 0.2 * s)
    m = jnp.max(s, axis=0, keepdims=True)
    e = jnp.exp(s - m)
    a = e / jnp.sum(e, axis=0, keepdims=True)
    na = jnp.sum(a[:, :, None] * nb, axis=0)

    h = jnp.maximum(
        jnp.dot(na, w1_ref[...], preferred_element_type=jnp.float32)
        + b1_ref[...], 0.0)
    refined = (jnp.dot(h, w2_ref[...], preferred_element_type=jnp.float32)
               + b2_ref[...])
    comb = jnp.maximum(
        jnp.dot(item, cwi_ref[...], preferred_element_type=jnp.float32)
        + jnp.dot(refined, cwr_ref[...], preferred_element_type=jnp.float32)
        + cb_ref[...], 0.0)
    score = (jnp.sum(user * owu_ref[...], axis=1)
             + jnp.sum(comb * owf_ref[...], axis=1) + ob_ref[0, 0])
    out_ref[...] = score[:, None]


def _tc_compute(nb3, item_emb, user_emb, wi, wn, ab, w1, b1, w2, b2,
                cwi, cwr, cb, owu, owf, ob):
    n_blocks = B // BBLK
    small = lambda shp: pl.BlockSpec(shp, lambda i: (0, 0))
    return pl.pallas_call(
        _tc_body,
        grid=(n_blocks,),
        in_specs=[
            pl.BlockSpec((NN, BBLK, D), lambda i: (0, i, 0)),
            pl.BlockSpec((BBLK, D), lambda i: (i, 0)),
            pl.BlockSpec((BBLK, D), lambda i: (i, 0)),
            small((1, D)), small((1, D)), small((1, 1)),
            small((D, D)), small((1, D)), small((D, D)), small((1, D)),
            small((D, D)), small((D, D)), small((1, D)),
            small((1, D)), small((1, D)), small((1, 1)),
        ],
        out_specs=pl.BlockSpec((BBLK, 1), lambda i: (i, 0)),
        out_shape=jax.ShapeDtypeStruct((B, 1), jnp.float32),
    )(nb3, item_emb, user_emb, wi, wn, ab, w1, b1, w2, b2,
      cwi, cwr, cb, owu, owf, ob)


def kernel(user_indices, item_indices, kg_adj_batch, user_table, item_table,
           entity_table, attn_W, attn_b, kg_W1, kg_b1, kg_W2, kg_b2,
           comb_W, comb_b, out_W, out_b):
    adj = jnp.maximum(kg_adj_batch, 0).astype(jnp.int32)
    adj_nm = adj.T.reshape(N_IDX_ROWS, IDX_W)
    ii = item_indices.astype(jnp.int32).reshape(B // IDX_W, IDX_W)
    ui = user_indices.astype(jnp.int32).reshape(B // IDX_W, IDX_W)

    nb_flat, item_emb, user_emb = _sc_gather_fn()(
        adj_nm, ii, ui, entity_table, item_table, user_table)
    nb3 = nb_flat.reshape(NN, B, D)

    wi = attn_W[:D, 0].reshape(1, D)
    wn = attn_W[D:, 0].reshape(1, D)
    ab = attn_b.reshape(1, 1)
    b1 = kg_b1.reshape(1, D)
    b2 = kg_b2.reshape(1, D)
    cwi = comb_W[:D]
    cwr = comb_W[D:]
    cb = comb_b.reshape(1, D)
    owu = out_W[:D, 0].reshape(1, D)
    owf = out_W[D:, 0].reshape(1, D)
    ob = out_b.reshape(1, 1)

    score = _tc_compute(nb3, item_emb, user_emb, wi, wn, ab, kg_W1, b1,
                        kg_W2, b2, cwi, cwr, cb, owu, owf, ob)
    return score[:, 0]
